# baseline (speedup 1.0000x reference)
import jax, jax.numpy as jnp
from jax import lax
from jax.experimental import pallas as pl
from jax.experimental.pallas import tpu as pltpu
from jax.experimental.pallas import tpu_sc as plsc

_SIZE = 100000
_SPLIT = 51200
_BC = 2048
_NW = 32
_RPW = 64
_CH = 512
_NCH = 95


def _tc_k(x_ref, out_ref):
    j = pl.program_id(0)

    @pl.when(j == 0)
    def _i():
        out_ref[:, :] = jnp.zeros((1, 1), jnp.float32)

    out_ref[:, :] += jnp.sum(x_ref[:, :]).reshape(1, 1)


def _sc_probe(x_hbm, out_hbm, buf0, buf1, acc_v, sem0, sem1):
    wid = lax.axis_index("s") * 2 + lax.axis_index("c")
    rows = pl.ds(wid * _RPW, _RPW)

    def start(i, buf, sem):
        return pltpu.async_copy(
            x_hbm.at[rows, pl.ds(_SPLIT + i * _CH, _CH)], buf, sem
        )

    def wait(i, buf, sem):
        pltpu.make_async_copy(
            x_hbm.at[rows, pl.ds(_SPLIT + i * _CH, _CH)], buf, sem
        ).wait()

    start(0, buf0, sem0)
    start(1, buf1, sem1)

    def body(k, carry):
        i0 = 2 * k
        wait(i0, buf0, sem0)

        @pl.when(i0 + 2 < _NCH)
        def _():
            start(i0 + 2, buf0, sem0)

        wait(i0 + 1, buf1, sem1)

        @pl.when(i0 + 3 < _NCH)
        def _():
            start(i0 + 3, buf1, sem1)

        return carry

    lax.fori_loop(0, _NCH // 2, body, 0)
    acc_v[...] = buf0[0, pl.ds(0, 16)]
    pltpu.sync_copy(acc_v, out_hbm.at[wid])


@jax.jit
def _run(x):
    n = x.shape[0]
    sc_run = pl.kernel(
        _sc_probe,
        out_type=jax.ShapeDtypeStruct((_NW, 16), jnp.float32),
        mesh=plsc.VectorSubcoreMesh(core_axis_name="c", subcore_axis_name="s"),
        scratch_types=[
            pltpu.VMEM((_RPW, _CH), jnp.float32),
            pltpu.VMEM((_RPW, _CH), jnp.float32),
            pltpu.VMEM((16,), jnp.float32),
            pltpu.SemaphoreType.DMA,
            pltpu.SemaphoreType.DMA,
        ],
    )
    tc_out = pl.pallas_call(
        _tc_k,
        grid=(_SPLIT // _BC,),
        in_specs=[pl.BlockSpec((n, _BC), lambda j: (0, j))],
        out_specs=pl.BlockSpec((1, 1), lambda j: (0, 0)),
        out_shape=jax.ShapeDtypeStruct((1, 1), jnp.float32),
    )(x)
    sc_out = sc_run(x)
    return tc_out[0, 0] + jnp.sum(sc_out)


def kernel(x, target, nwords):
    return _run(x.reshape(-1, _SIZE)) / nwords


# speedup vs baseline: 1.0370x; 1.0370x over previous
import jax, jax.numpy as jnp
import numpy as np
from jax import lax
from jax.experimental import pallas as pl
from jax.experimental.pallas import tpu as pltpu

_SIZE = 100000
_SMOOTHING = 0.1
_PAD_ID = 3

_EPS = np.float32(_SMOOTHING / (_SIZE - 2))
_TGT_COEFF = float(_EPS - np.float32(1.0 - _SMOOTHING))
_ROW_CONST = float(
    (_SIZE - 2) * (_EPS * np.log(_EPS))
    + np.float32(1.0 - _SMOOTHING) * np.log(np.float32(1.0 - _SMOOTHING))
)

_BR = 32


def _kl_kernel(t_ref, x_ref, out_ref):
    j = pl.program_id(0)

    t = t_ref[:, :]
    x = x_ref[:, :]
    row_ok = t != _PAD_ID

    rs = jnp.sum(x, axis=1, keepdims=True)
    main = -_EPS * jnp.sum(jnp.where(row_ok, rs, jnp.float32(0.0)))
    corr3 = _EPS * jnp.sum(
        jnp.where(row_ok, x[:, _PAD_ID : _PAD_ID + 1], jnp.float32(0.0))
    )
    count = jnp.sum(row_ok.astype(jnp.float32))

    lane = lax.broadcasted_iota(jnp.int32, (1, 128), 1)
    gacc = jnp.zeros((1, 128), jnp.float32)
    for r in range(_BR):
        idx = t_ref[r, 0]
        base = pl.multiple_of((idx // 128) * 128, 128)
        win = x_ref[r : r + 1, pl.ds(base, 128)]
        idx_sel = jnp.where(idx != _PAD_ID, idx - base, jnp.int32(-1))
        gacc = gacc + jnp.where(lane == idx_sel, win, jnp.float32(0.0))
    g = jnp.sum(gacc)

    contrib = main + corr3 + jnp.float32(_ROW_CONST) * count + _TGT_COEFF * g

    @pl.when(j == 0)
    def _init():
        out_ref[:, :] = jnp.zeros((1, 1), jnp.float32)

    out_ref[:, :] += contrib.reshape(1, 1)


@jax.jit
def _run(x, t):
    n = x.shape[0]
    out = pl.pallas_call(
        _kl_kernel,
        grid=(n // _BR,),
        in_specs=[
            pl.BlockSpec((_BR, 1), lambda j: (j, 0)),
            pl.BlockSpec((_BR, _SIZE), lambda j: (j, 0)),
        ],
        out_specs=pl.BlockSpec((1, 1), lambda j: (0, 0)),
        out_shape=jax.ShapeDtypeStruct((1, 1), jnp.float32),
    )(t, x)
    return out[0, 0]


def kernel(x, target, nwords):
    x2 = x.reshape(-1, _SIZE)
    t = target.reshape(-1).astype(jnp.int32)[:, None]
    return _run(x2, t) / nwords
